# Initial kernel scaffold; baseline (speedup 1.0000x reference)
import jax
import jax.numpy as jnp
from jax.experimental import pallas as pl


def kernel(x, W1, b1, W2, b2, gate_w, gate_b, bias):
    raise NotImplementedError("write your pallas kernel here")


# speedup vs baseline: 1.2070x; 1.2070x over previous
import functools

import jax
import jax.numpy as jnp
from jax.experimental import pallas as pl
from jax.experimental.pallas import tpu as pltpu

SEQ = 2048
D_MODEL = 1024
EXPERT_DIM = 2048
NUM_EXPERTS = 8
BT = 512


def _dense_body(x_ref, w1_ref, b1_ref, w2_ref, b2_ref, gw_ref, gb_ref,
                out_ref, comb_ref):
    e = pl.program_id(1)

    @pl.when(e == 0)
    def _gating():
        x = x_ref[...]
        logits = jnp.dot(x, gw_ref[...], preferred_element_type=jnp.float32)
        logits = logits + gb_ref[...]
        iota = jax.lax.broadcasted_iota(jnp.int32, (BT, NUM_EXPERTS), 1)
        m1 = jnp.max(logits, axis=-1, keepdims=True)
        is1 = logits == m1
        idx1 = jnp.min(jnp.where(is1, iota, NUM_EXPERTS), axis=-1,
                       keepdims=True)
        neg = jnp.float32(-jnp.inf)
        masked = jnp.where(iota == idx1, neg, logits)
        m2 = jnp.max(masked, axis=-1, keepdims=True)
        is2 = masked == m2
        idx2 = jnp.min(jnp.where(is2, iota, NUM_EXPERTS), axis=-1,
                       keepdims=True)
        e2 = jnp.exp(m2 - m1)
        s = 1.0 + e2
        w1 = 1.0 / s
        w2 = e2 / s
        comb_ref[...] = (jnp.where(iota == idx1, w1, 0.0)
                         + jnp.where(iota == idx2, w2, 0.0))

    x = x_ref[...]
    h = jnp.dot(x, w1_ref[0], preferred_element_type=jnp.float32) + b1_ref[0]
    h = jnp.maximum(h, 0.0)
    o = jnp.dot(h, w2_ref[0], preferred_element_type=jnp.float32) + b2_ref[0]
    comb = comb_ref[...]
    eio = jax.lax.broadcasted_iota(jnp.int32, (BT, NUM_EXPERTS), 1)
    w_e = jnp.sum(jnp.where(eio == e, comb, 0.0), axis=-1, keepdims=True)
    contrib = w_e * o

    @pl.when(e == 0)
    def _init():
        out_ref[...] = contrib

    @pl.when(e != 0)
    def _acc():
        out_ref[...] += contrib


@functools.partial(jax.jit, static_argnums=())
def _dense_moe(x_flat, W1, b1, W2, b2, gate_w, gb):
    T = x_flat.shape[0]
    grid = (T // BT, NUM_EXPERTS)
    return pl.pallas_call(
        _dense_body,
        grid=grid,
        in_specs=[
            pl.BlockSpec((BT, D_MODEL), lambda t, e: (t, 0)),
            pl.BlockSpec((1, D_MODEL, EXPERT_DIM), lambda t, e: (e, 0, 0)),
            pl.BlockSpec((1, 1, EXPERT_DIM), lambda t, e: (e, 0, 0)),
            pl.BlockSpec((1, EXPERT_DIM, D_MODEL), lambda t, e: (e, 0, 0)),
            pl.BlockSpec((1, 1, D_MODEL), lambda t, e: (e, 0, 0)),
            pl.BlockSpec((D_MODEL, NUM_EXPERTS), lambda t, e: (0, 0)),
            pl.BlockSpec((1, NUM_EXPERTS), lambda t, e: (0, 0)),
        ],
        out_specs=pl.BlockSpec((BT, D_MODEL), lambda t, e: (t, 0)),
        out_shape=jax.ShapeDtypeStruct((T, D_MODEL), jnp.float32),
        scratch_shapes=[pltpu.VMEM((BT, NUM_EXPERTS), jnp.float32)],
    )(x_flat, W1, b1.reshape(NUM_EXPERTS, 1, EXPERT_DIM), W2,
      b2.reshape(NUM_EXPERTS, 1, D_MODEL), gate_w, gb)


def kernel(x, W1, b1, W2, b2, gate_w, gate_b, bias):
    seq_len, batch_size, d_model = x.shape
    x_flat = x.reshape(-1, d_model)
    gb = (gate_b + bias).reshape(1, NUM_EXPERTS)
    out = _dense_moe(x_flat, W1, b1, W2, b2, gate_w, gb)
    return out.reshape(seq_len, batch_size, d_model)
